# baseline (speedup 1.0000x reference)
import jax
import jax.numpy as jnp
from jax import lax
from jax.experimental import pallas as pl
from jax.experimental.pallas import tpu as pltpu
from jax.experimental.pallas import tpu_sc as plsc

B = 16384
FC = 26
FN = 13
V = 100000
D = 16
FT = FC + FN

VP = 100096
GP = VP // 8
RT = FC * GP
NT_FULL = V // 128
TPB = 4
NBLK = 196

NC = 2
NS = 16
NW = NC * NS
RW = B // NW
CB = 128
NSUB = RW // CB

_BYPASS_K1 = False


def _k1_body(tab_hbm, tail_hbm, out_hbm,
             in0, in1, ou0, ou1, tl_v, isem0, isem1, osem0, osem1):
    c = lax.axis_index("c")
    s = lax.axis_index("s")
    wid = s * NC + c
    iota16 = lax.iota(jnp.int32, 16)

    kmax = jnp.where(wid < NBLK % NW, NBLK // NW + 1, NBLK // NW)
    total = FC * kmax
    half = FN * kmax

    def nxt(f, k):
        roll = (k + 1) >= kmax
        return f + roll.astype(jnp.int32), jnp.where(roll, 0, k + 1)

    def blk_vt(k):
        blk = wid + k * NW
        return jnp.where(blk == NBLK - 1, NT_FULL - TPB, blk * TPB)

    def fire_in(f, k, buf, sem):
        vt = blk_vt(k)
        pltpu.async_copy(
            tab_hbm.at[pl.ds(f * D, D), pl.ds(vt * 128, TPB * 128)], buf, sem)

    def transpose_unit(buf, obuf):
        @plsc.parallel_loop(0, TPB * 128, step=1, unroll=8)
        def _(d):
            col = (d + iota16) & (TPB * 128 - 1)
            vals = plsc.load_gather(buf, [iota16, col])
            row = lax.shift_right_logical(col, 3)
            ocol = lax.shift_left(col & 7, 4) + iota16
            plsc.store_scatter(obuf, [row, ocol], vals)

    def fire_out(f, k, obuf, sem):
        vt = blk_vt(k)
        pltpu.async_copy(
            obuf, out_hbm.at[pl.ds(f * GP + vt * D, TPB * D)], sem)

    fire_in(0, 0, in0, isem0)
    f1_0, k1_0 = nxt(0, 0)
    fire_in(f1_0, k1_0, in1, isem1)

    def body(i, carry):
        fa, ka = carry
        fb, kb = nxt(fa, ka)
        fc_, kc = nxt(fb, kb)
        fd, kd = nxt(fc_, kc)

        pltpu.make_async_copy(
            tab_hbm.at[pl.ds(0, D), pl.ds(0, TPB * 128)], in0, isem0).wait()

        @pl.when(i >= 1)
        def _():
            pltpu.make_async_copy(
                ou0, out_hbm.at[pl.ds(0, TPB * D)], osem0).wait()

        transpose_unit(in0, ou0)

        @pl.when(2 * i + 2 < total)
        def _():
            fire_in(fc_, kc, in0, isem0)

        fire_out(fa, ka, ou0, osem0)

        pltpu.make_async_copy(
            tab_hbm.at[pl.ds(0, D), pl.ds(0, TPB * 128)], in1, isem1).wait()

        @pl.when(i >= 1)
        def _():
            pltpu.make_async_copy(
                ou1, out_hbm.at[pl.ds(0, TPB * D)], osem1).wait()

        transpose_unit(in1, ou1)

        @pl.when(2 * i + 3 < total)
        def _():
            fire_in(fd, kd, in1, isem1)

        fire_out(fb, kb, ou1, osem1)

        return (fc_, kc)

    lax.fori_loop(0, half, body, (jnp.int32(0), jnp.int32(0)))
    pltpu.make_async_copy(ou0, out_hbm.at[pl.ds(0, TPB * D)], osem0).wait()
    pltpu.make_async_copy(ou1, out_hbm.at[pl.ds(0, TPB * D)], osem1).wait()

    @pl.when(wid < FC)
    def _():
        pltpu.sync_copy(tail_hbm.at[wid], tl_v)
        pltpu.sync_copy(tl_v, out_hbm.at[pl.ds(wid * GP + NT_FULL * D, 4)])


def _k2_body(tab_hbm, xcat_hbm, xnum_hbm, w_hbm, b_hbm, out_hbm,
             xcat_v, idx_v, rows_v, cblk_v, nblk_v, skw_v, xnum_v, wv, bv,
             gsem, csem, osem):
    c = lax.axis_index("c")
    s = lax.axis_index("s")
    wid = s * NC + c

    iota16 = lax.iota(jnp.int32, 16)
    iota26 = iota16 * FC
    iota13 = iota16 * FN

    pltpu.sync_copy(w_hbm, wv)
    pltpu.sync_copy(b_hbm, bv)

    def subchunk(sub, carry):
        b0 = wid * RW + sub * CB
        bt = wid * NSUB + sub

        pltpu.sync_copy(xcat_hbm.at[pl.ds(b0 * FC, CB * FC)], xcat_v)
        for f in range(FC):
            off = f * (V if _BYPASS_K1 else VP)
            for g in range(CB // 16):
                vals = plsc.load_gather(xcat_v, [iota26 + (g * 16 * FC + f)])
                idx_v[f, pl.ds(g * 16, 16)] = vals + off

        gcps = [pltpu.async_copy(tab_hbm.at[idx_v.at[f]], rows_v.at[f], gsem)
                for f in range(FC)]

        pltpu.sync_copy(xnum_hbm.at[pl.ds(b0 * FN, CB * FN)], xnum_v)

        def jbody(j, cr):
            wrow = wv[j]
            brow = bv[j]
            xs = [plsc.load_gather(xnum_v, [iota13 + (bs * 16 * FN) + j])
                  for bs in range(8)]
            for d in range(D):
                ws = jnp.broadcast_to(wrow[d], (16,))
                bss = jnp.broadcast_to(brow[d], (16,))
                for bs in range(8):
                    nblk_v[j, d // 8, d % 8, pl.ds(bs * 16, 16)] = (
                        xs[bs] * ws + bss)
            return cr

        lax.fori_loop(0, FN, jbody, 0)
        ncp = pltpu.async_copy(
            nblk_v, out_hbm.at[pl.ds(FC, FN), :, bt, :, :], osem)

        for cp in gcps:
            cp.wait()

        def tbody(f, cr):
            row = cr
            fsp = jnp.broadcast_to(f, (16,))
            for g in range(8):
                bidx = iota16 + g * 16
                for t in range(D):
                    dvec = (t + iota16) & (D - 1)
                    skw_v[t, :] = plsc.load_gather(rows_v, [fsp, bidx, dvec])
                for d in range(D):
                    srow = (d - iota16) & (D - 1)
                    cblk_v[row, d, pl.ds(g * 16, 16)] = plsc.load_gather(
                        skw_v, [srow, iota16])
            return cr + 1

        lax.fori_loop(0, FN, tbody, 0)
        cpa = [pltpu.async_copy(
            cblk_v.at[:, pl.ds(db * 8, 8), :],
            out_hbm.at[pl.ds(0, FN), db, bt, :, :], csem) for db in range(2)]
        for cp in cpa:
            cp.wait()

        lax.fori_loop(FN, FC, tbody, 0)
        cpb = [pltpu.async_copy(
            cblk_v.at[:, pl.ds(db * 8, 8), :],
            out_hbm.at[pl.ds(FN, FN), db, bt, :, :], csem) for db in range(2)]

        ncp.wait()
        for cp in cpb:
            cp.wait()
        return carry

    lax.fori_loop(0, NSUB, subchunk, 0)


@jax.jit
def _tokenize(x_num, x_cat, cat_tables, num_W, num_b):
    tab2d = cat_tables.transpose(0, 2, 1).reshape(FC * D, V)
    tailF = cat_tables[:, V - 32:, :].reshape(FC, 4, 128)
    xcat_flat = x_cat.astype(jnp.int32).reshape(B * FC)
    xnum_flat = x_num.reshape(B * FN)

    k1 = pl.kernel(
        _k1_body,
        out_type=jax.ShapeDtypeStruct((RT, 128), jnp.float32),
        mesh=plsc.VectorSubcoreMesh(core_axis_name="c", subcore_axis_name="s"),
        compiler_params=pltpu.CompilerParams(needs_layout_passes=False),
        scratch_types=[
            pltpu.VMEM((D, TPB * 128), jnp.float32),
            pltpu.VMEM((D, TPB * 128), jnp.float32),
            pltpu.VMEM((TPB * D, 128), jnp.float32),
            pltpu.VMEM((TPB * D, 128), jnp.float32),
            pltpu.VMEM((4, 128), jnp.float32),
            pltpu.SemaphoreType.DMA,
            pltpu.SemaphoreType.DMA,
            pltpu.SemaphoreType.DMA,
            pltpu.SemaphoreType.DMA,
        ],
    )
    tabF = k1(tab2d, tailF)
    tab16 = tabF.reshape(RT * 8, D)
    if _BYPASS_K1:
        tab16 = cat_tables.reshape(FC * V, D)

    k2 = pl.kernel(
        _k2_body,
        out_type=jax.ShapeDtypeStruct((FT, 2, B // CB, 8, 128), jnp.float32),
        mesh=plsc.VectorSubcoreMesh(core_axis_name="c", subcore_axis_name="s"),
        compiler_params=pltpu.CompilerParams(
            use_tc_tiling_on_sc=False, needs_layout_passes=False),
        scratch_types=[
            pltpu.VMEM((CB * FC,), jnp.int32),
            pltpu.VMEM((FC, CB), jnp.int32),
            pltpu.VMEM((FC, CB, D), jnp.float32),
            pltpu.VMEM((FN, D, 128), jnp.float32),
            pltpu.VMEM((FN, 2, 8, 128), jnp.float32),
            pltpu.VMEM((D, D), jnp.float32),
            pltpu.VMEM((CB * FN,), jnp.float32),
            pltpu.VMEM((FN, D), jnp.float32),
            pltpu.VMEM((FN, D), jnp.float32),
            pltpu.SemaphoreType.DMA,
            pltpu.SemaphoreType.DMA,
            pltpu.SemaphoreType.DMA,
        ],
    )
    out5 = k2(tab16, xcat_flat, xnum_flat, num_W, num_b)
    return out5.transpose(2, 4, 0, 1, 3).reshape(B, FT, D)


def kernel(x_num, x_cat, cat_tables, num_W, num_b):
    return _tokenize(x_num, x_cat, cat_tables, num_W, num_b)


# speedup vs baseline: 1.0294x; 1.0294x over previous
import jax
import jax.numpy as jnp
from jax import lax
from jax.experimental import pallas as pl
from jax.experimental.pallas import tpu as pltpu
from jax.experimental.pallas import tpu_sc as plsc

B = 16384
FC = 26
FN = 13
V = 100000
D = 16
FT = FC + FN

VP = 100096
GP = VP // 8
RT = FC * GP
NT_FULL = V // 128
TPB = 4
NBLK = 196

NC = 2
NS = 16
NW = NC * NS
RW = B // NW
CB = 128
NSUB = RW // CB

_BYPASS_K1 = False


def _k1_body(tab_hbm, tail_hbm, out_hbm,
             in0, in1, ou0, ou1, tl_v, isem0, isem1, osem0, osem1):
    c = lax.axis_index("c")
    s = lax.axis_index("s")
    wid = s * NC + c
    iota16 = lax.iota(jnp.int32, 16)

    kmax = jnp.where(wid < NBLK % NW, NBLK // NW + 1, NBLK // NW)
    total = FC * kmax
    half = FN * kmax

    def nxt(f, k):
        roll = (k + 1) >= kmax
        return f + roll.astype(jnp.int32), jnp.where(roll, 0, k + 1)

    def blk_vt(k):
        blk = wid + k * NW
        return jnp.where(blk == NBLK - 1, NT_FULL - TPB, blk * TPB)

    def fire_in(f, k, buf, sem):
        vt = blk_vt(k)
        pltpu.async_copy(
            tab_hbm.at[pl.ds(f * D, D), pl.ds(vt * 128, TPB * 128)], buf, sem)

    def transpose_unit(buf, obuf):
        @plsc.parallel_loop(0, TPB * 128, step=1, unroll=8)
        def _(d):
            col = (d + iota16) & (TPB * 128 - 1)
            vals = plsc.load_gather(buf, [iota16, col])
            row = lax.shift_right_logical(col, 3)
            ocol = lax.shift_left(col & 7, 4) + iota16
            plsc.store_scatter(obuf, [row, ocol], vals)

    def fire_out(f, k, obuf, sem):
        vt = blk_vt(k)
        pltpu.async_copy(
            obuf, out_hbm.at[pl.ds(f * GP + vt * D, TPB * D)], sem)

    fire_in(0, 0, in0, isem0)
    f1_0, k1_0 = nxt(0, 0)
    fire_in(f1_0, k1_0, in1, isem1)

    def body(i, carry):
        fa, ka = carry
        fb, kb = nxt(fa, ka)
        fc_, kc = nxt(fb, kb)
        fd, kd = nxt(fc_, kc)

        pltpu.make_async_copy(
            tab_hbm.at[pl.ds(0, D), pl.ds(0, TPB * 128)], in0, isem0).wait()

        @pl.when(i >= 1)
        def _():
            pltpu.make_async_copy(
                ou0, out_hbm.at[pl.ds(0, TPB * D)], osem0).wait()

        transpose_unit(in0, ou0)

        @pl.when(2 * i + 2 < total)
        def _():
            fire_in(fc_, kc, in0, isem0)

        fire_out(fa, ka, ou0, osem0)

        pltpu.make_async_copy(
            tab_hbm.at[pl.ds(0, D), pl.ds(0, TPB * 128)], in1, isem1).wait()

        @pl.when(i >= 1)
        def _():
            pltpu.make_async_copy(
                ou1, out_hbm.at[pl.ds(0, TPB * D)], osem1).wait()

        transpose_unit(in1, ou1)

        @pl.when(2 * i + 3 < total)
        def _():
            fire_in(fd, kd, in1, isem1)

        fire_out(fb, kb, ou1, osem1)

        return (fc_, kc)

    lax.fori_loop(0, half, body, (jnp.int32(0), jnp.int32(0)))
    pltpu.make_async_copy(ou0, out_hbm.at[pl.ds(0, TPB * D)], osem0).wait()
    pltpu.make_async_copy(ou1, out_hbm.at[pl.ds(0, TPB * D)], osem1).wait()

    @pl.when(wid < FC)
    def _():
        pltpu.sync_copy(tail_hbm.at[wid], tl_v)
        pltpu.sync_copy(tl_v, out_hbm.at[pl.ds(wid * GP + NT_FULL * D, 4)])


def _k2_body(tab_hbm, xcat_hbm, xnum_hbm, w_hbm, b_hbm, out_hbm,
             xcat_v, idx_v, rowsA, rowsB, cblkA, cblkB, skw_v, xnum_v, wv, bv,
             gsem, csem, osem):
    c = lax.axis_index("c")
    s = lax.axis_index("s")
    wid = s * NC + c

    iota16 = lax.iota(jnp.int32, 16)
    iota26 = iota16 * FC
    iota13 = iota16 * FN

    pltpu.sync_copy(w_hbm, wv)
    pltpu.sync_copy(b_hbm, bv)

    def subchunk(sub, carry):
        b0 = wid * RW + sub * CB
        bt = wid * NSUB + sub

        pltpu.sync_copy(xcat_hbm.at[pl.ds(b0 * FC, CB * FC)], xcat_v)
        for f in range(FC):
            off = f * (V if _BYPASS_K1 else VP)
            for g in range(CB // 16):
                vals = plsc.load_gather(xcat_v, [iota26 + (g * 16 * FC + f)])
                idx_v[f, pl.ds(g * 16, 16)] = vals + off

        hA = [pltpu.async_copy(tab_hbm.at[idx_v.at[f]], rowsA.at[f], gsem)
              for f in range(FN)]

        pltpu.sync_copy(xnum_hbm.at[pl.ds(b0 * FN, CB * FN)], xnum_v)

        def jbody(j, cr):
            wrow = wv[j]
            brow = bv[j]
            xs = [plsc.load_gather(xnum_v, [iota13 + (bs * 16 * FN) + j])
                  for bs in range(8)]
            for d in range(D):
                ws = jnp.broadcast_to(wrow[d], (16,))
                bss = jnp.broadcast_to(brow[d], (16,))
                for bs in range(8):
                    cblkB[j, d, pl.ds(bs * 16, 16)] = xs[bs] * ws + bss
            return cr

        lax.fori_loop(0, FN, jbody, 0)
        ncp = [pltpu.async_copy(
            cblkB.at[:, pl.ds(db * 8, 8), :],
            out_hbm.at[pl.ds(FC, FN), db, bt, :, :], osem) for db in range(2)]

        for cp in hA:
            cp.wait()
        hB = [pltpu.async_copy(
            tab_hbm.at[idx_v.at[FN + f]], rowsB.at[f], gsem)
            for f in range(FN)]

        def trans(slot, rows_ref, blk_ref):
            ssp = jnp.broadcast_to(slot, (16,))
            for g in range(8):
                bidx = iota16 + g * 16
                for t in range(D):
                    dvec = (t + iota16) & (D - 1)
                    skw_v[t, :] = plsc.load_gather(rows_ref, [ssp, bidx, dvec])
                for d in range(D):
                    srow = (d - iota16) & (D - 1)
                    blk_ref[slot, d, pl.ds(g * 16, 16)] = plsc.load_gather(
                        skw_v, [srow, iota16])

        def ta(f, cr):
            trans(f, rowsA, cblkA)
            return cr

        lax.fori_loop(0, FN, ta, 0)
        cpa = [pltpu.async_copy(
            cblkA.at[:, pl.ds(db * 8, 8), :],
            out_hbm.at[pl.ds(0, FN), db, bt, :, :], csem) for db in range(2)]

        for cp in hB:
            cp.wait()
        for cp in ncp:
            cp.wait()

        def tb(f, cr):
            trans(f, rowsB, cblkB)
            return cr

        lax.fori_loop(0, FN, tb, 0)
        cpb = [pltpu.async_copy(
            cblkB.at[:, pl.ds(db * 8, 8), :],
            out_hbm.at[pl.ds(FN, FN), db, bt, :, :], csem) for db in range(2)]

        for cp in cpa:
            cp.wait()
        for cp in cpb:
            cp.wait()
        return carry

    lax.fori_loop(0, NSUB, subchunk, 0)


@jax.jit
def _tokenize(x_num, x_cat, cat_tables, num_W, num_b):
    tab2d = cat_tables.transpose(0, 2, 1).reshape(FC * D, V)
    tailF = cat_tables[:, V - 32:, :].reshape(FC, 4, 128)
    xcat_flat = x_cat.astype(jnp.int32).reshape(B * FC)
    xnum_flat = x_num.reshape(B * FN)

    k1 = pl.kernel(
        _k1_body,
        out_type=jax.ShapeDtypeStruct((RT, 128), jnp.float32),
        mesh=plsc.VectorSubcoreMesh(core_axis_name="c", subcore_axis_name="s"),
        compiler_params=pltpu.CompilerParams(needs_layout_passes=False),
        scratch_types=[
            pltpu.VMEM((D, TPB * 128), jnp.float32),
            pltpu.VMEM((D, TPB * 128), jnp.float32),
            pltpu.VMEM((TPB * D, 128), jnp.float32),
            pltpu.VMEM((TPB * D, 128), jnp.float32),
            pltpu.VMEM((4, 128), jnp.float32),
            pltpu.SemaphoreType.DMA,
            pltpu.SemaphoreType.DMA,
            pltpu.SemaphoreType.DMA,
            pltpu.SemaphoreType.DMA,
        ],
    )
    tabF = k1(tab2d, tailF)
    tab16 = tabF.reshape(RT * 8, D)
    if _BYPASS_K1:
        tab16 = cat_tables.reshape(FC * V, D)

    k2 = pl.kernel(
        _k2_body,
        out_type=jax.ShapeDtypeStruct((FT, 2, B // CB, 8, 128), jnp.float32),
        mesh=plsc.VectorSubcoreMesh(core_axis_name="c", subcore_axis_name="s"),
        compiler_params=pltpu.CompilerParams(
            use_tc_tiling_on_sc=False, needs_layout_passes=False),
        scratch_types=[
            pltpu.VMEM((CB * FC,), jnp.int32),
            pltpu.VMEM((FC, CB), jnp.int32),
            pltpu.VMEM((FN, CB, D), jnp.float32),
            pltpu.VMEM((FN, CB, D), jnp.float32),
            pltpu.VMEM((FN, D, 128), jnp.float32),
            pltpu.VMEM((FN, D, 128), jnp.float32),
            pltpu.VMEM((D, D), jnp.float32),
            pltpu.VMEM((CB * FN,), jnp.float32),
            pltpu.VMEM((FN, D), jnp.float32),
            pltpu.VMEM((FN, D), jnp.float32),
            pltpu.SemaphoreType.DMA,
            pltpu.SemaphoreType.DMA,
            pltpu.SemaphoreType.DMA,
        ],
    )
    out5 = k2(tab16, xcat_flat, xnum_flat, num_W, num_b)
    return out5.transpose(2, 4, 0, 1, 3).reshape(B, FT, D)


def kernel(x_num, x_cat, cat_tables, num_W, num_b):
    return _tokenize(x_num, x_cat, cat_tables, num_W, num_b)


# speedup vs baseline: 1.0470x; 1.0171x over previous
import jax
import jax.numpy as jnp
from jax import lax
from jax.experimental import pallas as pl
from jax.experimental.pallas import tpu as pltpu
from jax.experimental.pallas import tpu_sc as plsc

B = 16384
FC = 26
FN = 13
V = 100000
D = 16
FT = FC + FN

VP = 100096
GP = VP // 8
RT = FC * GP
NT_FULL = V // 128
TPB = 8
NBLK = 98

NC = 2
NS = 16
NW = NC * NS
RW = B // NW
CB = 128
NSUB = RW // CB

_BYPASS_K1 = False


def _k1_body(tab_hbm, tail_hbm, out_hbm,
             in0, in1, ou0, ou1, tl_v, isem0, isem1, osem0, osem1):
    c = lax.axis_index("c")
    s = lax.axis_index("s")
    wid = s * NC + c
    iota16 = lax.iota(jnp.int32, 16)

    kmax = jnp.where(wid < NBLK % NW, NBLK // NW + 1, NBLK // NW)
    total = FC * kmax
    half = FN * kmax

    def nxt(f, k):
        roll = (k + 1) >= kmax
        return f + roll.astype(jnp.int32), jnp.where(roll, 0, k + 1)

    def blk_vt(k):
        blk = wid + k * NW
        return jnp.where(blk == NBLK - 1, NT_FULL - TPB, blk * TPB)

    def fire_in(f, k, buf, sem):
        vt = blk_vt(k)
        pltpu.async_copy(
            tab_hbm.at[pl.ds(f * D, D), pl.ds(vt * 128, TPB * 128)], buf, sem)

    def transpose_unit(buf, obuf):
        @plsc.parallel_loop(0, TPB * 128, step=1, unroll=8)
        def _(d):
            col = (d + iota16) & (TPB * 128 - 1)
            vals = plsc.load_gather(buf, [iota16, col])
            row = lax.shift_right_logical(col, 3)
            ocol = lax.shift_left(col & 7, 4) + iota16
            plsc.store_scatter(obuf, [row, ocol], vals)

    def fire_out(f, k, obuf, sem):
        vt = blk_vt(k)
        pltpu.async_copy(
            obuf, out_hbm.at[pl.ds(f * GP + vt * D, TPB * D)], sem)

    fire_in(0, 0, in0, isem0)
    f1_0, k1_0 = nxt(0, 0)
    fire_in(f1_0, k1_0, in1, isem1)

    def body(i, carry):
        fa, ka = carry
        fb, kb = nxt(fa, ka)
        fc_, kc = nxt(fb, kb)
        fd, kd = nxt(fc_, kc)

        pltpu.make_async_copy(
            tab_hbm.at[pl.ds(0, D), pl.ds(0, TPB * 128)], in0, isem0).wait()

        @pl.when(i >= 1)
        def _():
            pltpu.make_async_copy(
                ou0, out_hbm.at[pl.ds(0, TPB * D)], osem0).wait()

        transpose_unit(in0, ou0)

        @pl.when(2 * i + 2 < total)
        def _():
            fire_in(fc_, kc, in0, isem0)

        fire_out(fa, ka, ou0, osem0)

        pltpu.make_async_copy(
            tab_hbm.at[pl.ds(0, D), pl.ds(0, TPB * 128)], in1, isem1).wait()

        @pl.when(i >= 1)
        def _():
            pltpu.make_async_copy(
                ou1, out_hbm.at[pl.ds(0, TPB * D)], osem1).wait()

        transpose_unit(in1, ou1)

        @pl.when(2 * i + 3 < total)
        def _():
            fire_in(fd, kd, in1, isem1)

        fire_out(fb, kb, ou1, osem1)

        return (fc_, kc)

    lax.fori_loop(0, half, body, (jnp.int32(0), jnp.int32(0)))
    pltpu.make_async_copy(ou0, out_hbm.at[pl.ds(0, TPB * D)], osem0).wait()
    pltpu.make_async_copy(ou1, out_hbm.at[pl.ds(0, TPB * D)], osem1).wait()

    @pl.when(wid < FC)
    def _():
        pltpu.sync_copy(tail_hbm.at[wid], tl_v)
        pltpu.sync_copy(tl_v, out_hbm.at[pl.ds(wid * GP + NT_FULL * D, 4)])


def _k2_body(tab_hbm, xcat_hbm, xnum_hbm, w_hbm, b_hbm, out_hbm,
             xcat_v, idx_v, rowsA, rowsB, cblkA, cblkB, skw_v, xnum_v, wv, bv,
             gsem, csem, osem):
    c = lax.axis_index("c")
    s = lax.axis_index("s")
    wid = s * NC + c

    iota16 = lax.iota(jnp.int32, 16)
    iota26 = iota16 * FC
    iota13 = iota16 * FN

    pltpu.sync_copy(w_hbm, wv)
    pltpu.sync_copy(b_hbm, bv)

    def subchunk(sub, carry):
        b0 = wid * RW + sub * CB
        bt = wid * NSUB + sub

        pltpu.sync_copy(xcat_hbm.at[pl.ds(b0 * FC, CB * FC)], xcat_v)
        for f in range(FC):
            off = f * (V if _BYPASS_K1 else VP)
            for g in range(CB // 16):
                vals = plsc.load_gather(xcat_v, [iota26 + (g * 16 * FC + f)])
                idx_v[f, pl.ds(g * 16, 16)] = vals + off

        hA = [pltpu.async_copy(tab_hbm.at[idx_v.at[f]], rowsA.at[f], gsem)
              for f in range(FN)]

        pltpu.sync_copy(xnum_hbm.at[pl.ds(b0 * FN, CB * FN)], xnum_v)

        def jbody(j, cr):
            wrow = wv[j]
            brow = bv[j]
            xs = [plsc.load_gather(xnum_v, [iota13 + (bs * 16 * FN) + j])
                  for bs in range(8)]
            for d in range(D):
                ws = jnp.broadcast_to(wrow[d], (16,))
                bss = jnp.broadcast_to(brow[d], (16,))
                for bs in range(8):
                    cblkB[j, d, pl.ds(bs * 16, 16)] = xs[bs] * ws + bss
            return cr

        lax.fori_loop(0, FN, jbody, 0)
        ncp = [pltpu.async_copy(
            cblkB.at[:, pl.ds(db * 8, 8), :],
            out_hbm.at[pl.ds(FC, FN), db, bt, :, :], osem) for db in range(2)]

        for cp in hA:
            cp.wait()
        hB = [pltpu.async_copy(
            tab_hbm.at[idx_v.at[FN + f]], rowsB.at[f], gsem)
            for f in range(FN)]

        def trans(slot, rows_ref, blk_ref):
            ssp = jnp.broadcast_to(slot, (16,))
            for g in range(8):
                bidx = iota16 + g * 16
                for t in range(D):
                    dvec = (t + iota16) & (D - 1)
                    skw_v[t, :] = plsc.load_gather(rows_ref, [ssp, bidx, dvec])
                for d in range(D):
                    srow = (d - iota16) & (D - 1)
                    blk_ref[slot, d, pl.ds(g * 16, 16)] = plsc.load_gather(
                        skw_v, [srow, iota16])

        def ta(f, cr):
            trans(f, rowsA, cblkA)
            return cr

        lax.fori_loop(0, FN, ta, 0)
        cpa = [pltpu.async_copy(
            cblkA.at[:, pl.ds(db * 8, 8), :],
            out_hbm.at[pl.ds(0, FN), db, bt, :, :], csem) for db in range(2)]

        for cp in hB:
            cp.wait()
        for cp in ncp:
            cp.wait()

        def tb(f, cr):
            trans(f, rowsB, cblkB)
            return cr

        lax.fori_loop(0, FN, tb, 0)
        cpb = [pltpu.async_copy(
            cblkB.at[:, pl.ds(db * 8, 8), :],
            out_hbm.at[pl.ds(FN, FN), db, bt, :, :], csem) for db in range(2)]

        for cp in cpa:
            cp.wait()
        for cp in cpb:
            cp.wait()
        return carry

    lax.fori_loop(0, NSUB, subchunk, 0)


@jax.jit
def _tokenize(x_num, x_cat, cat_tables, num_W, num_b):
    tab2d = cat_tables.transpose(0, 2, 1).reshape(FC * D, V)
    tailF = cat_tables[:, V - 32:, :].reshape(FC, 4, 128)
    xcat_flat = x_cat.astype(jnp.int32).reshape(B * FC)
    xnum_flat = x_num.reshape(B * FN)

    k1 = pl.kernel(
        _k1_body,
        out_type=jax.ShapeDtypeStruct((RT, 128), jnp.float32),
        mesh=plsc.VectorSubcoreMesh(core_axis_name="c", subcore_axis_name="s"),
        compiler_params=pltpu.CompilerParams(needs_layout_passes=False),
        scratch_types=[
            pltpu.VMEM((D, TPB * 128), jnp.float32),
            pltpu.VMEM((D, TPB * 128), jnp.float32),
            pltpu.VMEM((TPB * D, 128), jnp.float32),
            pltpu.VMEM((TPB * D, 128), jnp.float32),
            pltpu.VMEM((4, 128), jnp.float32),
            pltpu.SemaphoreType.DMA,
            pltpu.SemaphoreType.DMA,
            pltpu.SemaphoreType.DMA,
            pltpu.SemaphoreType.DMA,
        ],
    )
    tabF = k1(tab2d, tailF)
    tab16 = tabF.reshape(RT * 8, D)
    if _BYPASS_K1:
        tab16 = cat_tables.reshape(FC * V, D)

    k2 = pl.kernel(
        _k2_body,
        out_type=jax.ShapeDtypeStruct((FT, 2, B // CB, 8, 128), jnp.float32),
        mesh=plsc.VectorSubcoreMesh(core_axis_name="c", subcore_axis_name="s"),
        compiler_params=pltpu.CompilerParams(
            use_tc_tiling_on_sc=False, needs_layout_passes=False),
        scratch_types=[
            pltpu.VMEM((CB * FC,), jnp.int32),
            pltpu.VMEM((FC, CB), jnp.int32),
            pltpu.VMEM((FN, CB, D), jnp.float32),
            pltpu.VMEM((FN, CB, D), jnp.float32),
            pltpu.VMEM((FN, D, 128), jnp.float32),
            pltpu.VMEM((FN, D, 128), jnp.float32),
            pltpu.VMEM((D, D), jnp.float32),
            pltpu.VMEM((CB * FN,), jnp.float32),
            pltpu.VMEM((FN, D), jnp.float32),
            pltpu.VMEM((FN, D), jnp.float32),
            pltpu.SemaphoreType.DMA,
            pltpu.SemaphoreType.DMA,
            pltpu.SemaphoreType.DMA,
        ],
    )
    out5 = k2(tab16, xcat_flat, xnum_flat, num_W, num_b)
    return out5.transpose(2, 4, 0, 1, 3).reshape(B, FT, D)


def kernel(x_num, x_cat, cat_tables, num_W, num_b):
    return _tokenize(x_num, x_cat, cat_tables, num_W, num_b)
